# Initial kernel scaffold; baseline (speedup 1.0000x reference)
import jax
import jax.numpy as jnp
from jax.experimental import pallas as pl


def kernel(x, G_rows, G_cols, G_vals, L_rows, L_cols, L_vals, F2V_rows, F2V_cols, F2V_vals, NS, EW, coeffs, bias):
    raise NotImplementedError("write your pallas kernel here")


# speedup vs baseline: 20.5912x; 20.5912x over previous
import functools

import jax
import jax.numpy as jnp
import numpy as np
from jax import lax
from jax.experimental import pallas as pl
from jax.experimental.pallas import tpu as pltpu
from jax.experimental.pallas import tpu_sc as plsc

NV = 40962
NF = 81920
B = 2
C = 64
NVP = 41472
FW = 16
NITEMS = 8
BLK = 1024
NCORES = 2
NSUB = 16

_GDN = jax.lax.GatherDimensionNumbers(
    offset_dims=(), collapsed_slice_dims=(0,), start_index_map=(0,))


def _splat(v16, zeros16, i):
    return lax.gather(v16, (zeros16 + i)[:, None], _GDN, (1,),
                      mode=lax.GatherScatterMode.PROMISE_IN_BOUNDS)


def _make_spmm(nnz, r_out, n_win, zc):
    win_r = r_out // n_win
    items_per_sc = (n_win * NITEMS) // NCORES
    tile_rows = win_r // NSUB
    tile_nnz = nnz // NSUB
    n_blk = tile_nnz // BLK
    n_row = BLK // 128

    mesh = plsc.VectorSubcoreMesh(core_axis_name="c", subcore_axis_name="s",
                                  num_cores=NCORES, num_subcores=NSUB)

    @functools.partial(
        pl.kernel,
        out_type=jax.ShapeDtypeStruct((r_out, NITEMS, FW), jnp.float32),
        mesh=mesh,
        scratch_types=[
            pltpu.VMEM_SHARED((win_r + 16, FW), jnp.float32),
            pltpu.VMEM((BLK,), jnp.int32),
            pltpu.VMEM((BLK,), jnp.int32),
            pltpu.VMEM((BLK,), jnp.float32),
            pltpu.VMEM((n_row, 128), jnp.int32),
            pltpu.VMEM((n_row, 128), jnp.int32),
            pltpu.VMEM((BLK, FW), jnp.float32),
            pltpu.VMEM((zc, FW), jnp.float32),
            pltpu.SemaphoreType.DMA,
        ],
        compiler_params=pltpu.CompilerParams(use_tc_tiling_on_sc=False),
    )
    def spmm(x_hbm, rows_hbm, cols_hbm, vals_hbm, out_hbm,
             acc, rows_v, cols_v, vals_v, idxb, relb, gbuf, zbuf, sem):
        cid = lax.axis_index("c")
        sid = lax.axis_index("s")

        def zinit(zr):
            zbuf[zr, :] = jnp.zeros((FW,), jnp.float32)
        pl.loop(0, zc)(zinit)

        def item_body(it):
            g_it = cid * items_per_sc + it
            win = g_it // NITEMS
            fch = g_it % NITEMS
            wlo = win * win_r

            def zero_blk(z):
                pltpu.sync_copy(zbuf, acc.at[pl.ds(sid * tile_rows + z * zc, zc)])
            pl.loop(0, tile_rows // zc)(zero_blk)
            plsc.subcore_barrier()

            def blk_body(b):
                off = sid * tile_nnz + b * BLK
                d1 = pltpu.async_copy(rows_hbm.at[pl.ds(off, BLK)], rows_v, sem)
                d2 = pltpu.async_copy(cols_hbm.at[pl.ds(off, BLK)], cols_v, sem)
                d3 = pltpu.async_copy(vals_hbm.at[pl.ds(off, BLK)], vals_v, sem)
                d1.wait(); d2.wait(); d3.wait()

                def idx_body(r):
                    def idx16(q):
                        j = r * 8 + q
                        c16 = cols_v[pl.ds(j * 16, 16)]
                        r16 = rows_v[pl.ds(j * 16, 16)]
                        inwin = (r16 >= wlo) & (r16 < wlo + win_r)
                        rel = jnp.where(inwin, r16 - wlo, win_r)
                        v16 = vals_v[pl.ds(j * 16, 16)]
                        vals_v[pl.ds(j * 16, 16)] = jnp.where(
                            inwin, v16, jnp.zeros((16,), jnp.float32))
                        idxb[r, pl.ds(q * 16, 16)] = c16 * NITEMS + fch
                        relb[r, pl.ds(q * 16, 16)] = rel
                    pl.loop(0, 8)(idx16)
                pl.loop(0, n_row)(idx_body)

                gd = [pltpu.async_copy(x_hbm.at[idxb.at[r]],
                                       gbuf.at[pl.ds(r * 128, 128)], sem)
                      for r in range(n_row)]
                for d in gd:
                    d.wait()

                zeros16 = lax.broadcasted_iota(jnp.int32, (16,), 0) * 0
                def scale16(j):
                    v16 = vals_v[pl.ds(j * 16, 16)]
                    for i in range(16):
                        e = j * 16 + i
                        gbuf[e, :] = gbuf[e, :] * _splat(v16, zeros16, i)
                pl.loop(0, BLK // 16)(scale16)

                sd = [pltpu.async_copy(gbuf.at[pl.ds(r * 128, 128)],
                                       acc.at[relb.at[r]], sem, add=True)
                      for r in range(n_row)]
                for d in sd:
                    d.wait()
            pl.loop(0, n_blk)(blk_body)
            plsc.subcore_barrier()

            pltpu.sync_copy(
                acc.at[pl.ds(sid * tile_rows, tile_rows)],
                out_hbm.at[pl.ds(wlo + sid * tile_rows, tile_rows), fch, :])
            plsc.subcore_barrier()
        pl.loop(0, items_per_sc)(item_body)

    return spmm


_spmm_g = _make_spmm(737280, 3 * NF, 3, 512)
_spmm_nv = _make_spmm(294912, NVP, 1, 432)
_spmm_f2v = _make_spmm(245760, NVP, 1, 432)

_NF_BLK = 512
_NV_BLK = 512


def _k2_body(y3, ewt, nst, ew_o, ns_o):
    y = y3[...]
    ew = ewt[...]
    ns = nst[...]
    ew_o[...] = (y[0] * ew[0][:, None] + y[1] * ew[1][:, None]
                 + y[2] * ew[2][:, None])
    ns_o[...] = (y[0] * ns[0][:, None] + y[1] * ns[1][:, None]
                 + y[2] * ns[2][:, None])


def _k4_body(xv, fl, fe, fn, m2, b2, out):
    acc = jnp.dot(xv[...], m2[0], preferred_element_type=jnp.float32)
    acc += jnp.dot(fl[...], m2[1], preferred_element_type=jnp.float32)
    acc += jnp.dot(fe[...], m2[2], preferred_element_type=jnp.float32)
    acc += jnp.dot(fn[...], m2[3], preferred_element_type=jnp.float32)
    out[...] = acc + b2[...]


def kernel(x, G_rows, G_cols, G_vals, L_rows, L_cols, L_vals,
           F2V_rows, F2V_cols, F2V_vals, NS, EW, coeffs, bias):
    x_vm = jnp.pad(x.reshape(B * C, NV).T, ((0, NVP - NV), (0, 0)))
    x_flat = x_vm.reshape(NVP * NITEMS, FW)
    lp = 294912 - L_rows.shape[0]
    l_rows = jnp.pad(L_rows, (0, lp))
    l_cols = jnp.pad(L_cols, (0, lp))
    l_vals = jnp.pad(L_vals, (0, lp))

    y3 = _spmm_g(x_flat, G_rows, G_cols, G_vals).reshape(3, NF, 128)

    f_lap = _spmm_nv(x_flat, l_rows, l_cols, l_vals).reshape(NVP, 128)

    ewt = EW.T
    nst = NS.T
    gf_ew, gf_ns = pl.pallas_call(
        _k2_body,
        grid=(NF // _NF_BLK,),
        in_specs=[
            pl.BlockSpec((3, _NF_BLK, 128), lambda i: (0, i, 0)),
            pl.BlockSpec((3, _NF_BLK), lambda i: (0, i)),
            pl.BlockSpec((3, _NF_BLK), lambda i: (0, i)),
        ],
        out_specs=[
            pl.BlockSpec((_NF_BLK, 128), lambda i: (i, 0)),
            pl.BlockSpec((_NF_BLK, 128), lambda i: (i, 0)),
        ],
        out_shape=[jax.ShapeDtypeStruct((NF, 128), jnp.float32)] * 2,
    )(y3, ewt, nst)

    f_ew = _spmm_f2v(gf_ew.reshape(NF * NITEMS, FW),
                     F2V_rows, F2V_cols, F2V_vals).reshape(NVP, 128)
    f_ns = _spmm_f2v(gf_ns.reshape(NF * NITEMS, FW),
                     F2V_rows, F2V_cols, F2V_vals).reshape(NVP, 128)

    a = jnp.transpose(coeffs, (2, 1, 0))
    m2 = jnp.zeros((4, 128, 128), jnp.float32)
    m2 = m2.at[:, :C, :C].set(a).at[:, C:, C:].set(a)
    b2 = jnp.concatenate([bias, bias])[None]
    out = pl.pallas_call(
        _k4_body,
        grid=(NVP // _NV_BLK,),
        in_specs=[
            pl.BlockSpec((_NV_BLK, 128), lambda i: (i, 0)),
            pl.BlockSpec((_NV_BLK, 128), lambda i: (i, 0)),
            pl.BlockSpec((_NV_BLK, 128), lambda i: (i, 0)),
            pl.BlockSpec((_NV_BLK, 128), lambda i: (i, 0)),
            pl.BlockSpec((4, 128, 128), lambda i: (0, 0, 0)),
            pl.BlockSpec((1, 128), lambda i: (0, 0)),
        ],
        out_specs=pl.BlockSpec((_NV_BLK, 128), lambda i: (i, 0)),
        out_shape=jax.ShapeDtypeStruct((NVP, 128), jnp.float32),
    )(x_vm, f_lap, f_ew, f_ns, m2, b2)

    return out[:NV].T.reshape(B, C, NV)


# speedup vs baseline: 45.0448x; 2.1876x over previous
import functools

import jax
import jax.numpy as jnp
import numpy as np
from jax import lax
from jax.experimental import pallas as pl
from jax.experimental.pallas import tpu as pltpu
from jax.experimental.pallas import tpu_sc as plsc

NV = 40962
NF = 81920
B = 2
C = 64
NVP = 41472
FW = 16
NITEMS = 8
BLK = 1024
NCORES = 2
NSUB = 16

_GDN = jax.lax.GatherDimensionNumbers(
    offset_dims=(), collapsed_slice_dims=(0,), start_index_map=(0,))


def _splat(v16, zeros16, i):
    return lax.gather(v16, (zeros16 + i)[:, None], _GDN, (1,),
                      mode=lax.GatherScatterMode.PROMISE_IN_BOUNDS)


def _prefix16(inwin, iota16):
    m = jnp.where(inwin, iota16 * 0 + 1, iota16 * 0)
    for k in (1, 2, 4, 8):
        sh = lax.gather(m, jnp.maximum(iota16 - k, 0)[:, None], _GDN, (1,),
                        mode=lax.GatherScatterMode.PROMISE_IN_BOUNDS)
        m = m + jnp.where(iota16 >= k, sh, iota16 * 0)
    return m


def _make_spmm2(nnz, r_out, n_win, zc):
    win_r = r_out // n_win
    tile_rows = win_r // NSUB
    tile_nnz = nnz // NSUB
    n_blk = tile_nnz // BLK
    CBUF = 1312

    mesh = plsc.VectorSubcoreMesh(core_axis_name="c", subcore_axis_name="s",
                                  num_cores=NCORES, num_subcores=NSUB)

    @functools.partial(
        pl.kernel,
        out_type=jax.ShapeDtypeStruct((r_out, 2, 64), jnp.float32),
        mesh=mesh,
        scratch_types=[
            pltpu.VMEM_SHARED((win_r + 16, 64), jnp.float32),
            pltpu.VMEM((BLK,), jnp.int32),
            pltpu.VMEM((BLK,), jnp.int32),
            pltpu.VMEM((BLK,), jnp.float32),
            pltpu.VMEM((CBUF,), jnp.int32),
            pltpu.VMEM((CBUF,), jnp.int32),
            pltpu.VMEM((CBUF,), jnp.float32),
            pltpu.VMEM((1, 128), jnp.int32),
            pltpu.VMEM((1, 128), jnp.int32),
            pltpu.VMEM((128, 64), jnp.float32),
            pltpu.VMEM((zc, 64), jnp.float32),
            pltpu.SemaphoreType.DMA,
        ],
        compiler_params=pltpu.CompilerParams(use_tc_tiling_on_sc=False, needs_layout_passes=False),
    )
    def spmm(x_hbm, rows_hbm, cols_hbm, vals_hbm, out_hbm,
             acc, rows_v, cols_v, vals_v, crel, cgid, cval, relb, gidb, gbuf,
             zbuf, sem):
        cid = lax.axis_index("c")
        sid = lax.axis_index("s")
        iota16 = lax.broadcasted_iota(jnp.int32, (16,), 0)
        zeros16 = iota16 * 0

        def zinit(zr):
            for q in range(4):
                zbuf[zr, pl.ds(q * 16, 16)] = jnp.zeros((16,), jnp.float32)
        pl.loop(0, zc)(zinit)

        def drain(off):
            ng = off // 128

            def group(g, _):
                for i in range(8):
                    relb[0, pl.ds(i * 16, 16)] = crel[pl.ds(g * 128 + i * 16, 16)]
                    gidb[0, pl.ds(i * 16, 16)] = cgid[pl.ds(g * 128 + i * 16, 16)]
                pltpu.async_copy(x_hbm.at[gidb.at[0]], gbuf, sem).wait()

                def scale(j, _):
                    v16 = cval[pl.ds(g * 128 + j * 16, 16)]
                    for i in range(16):
                        e = j * 16 + i
                        sp = _splat(v16, zeros16, i)
                        for q in range(4):
                            gbuf[e, pl.ds(q * 16, 16)] = (
                                gbuf[e, pl.ds(q * 16, 16)] * sp)
                    return 0
                lax.fori_loop(0, 8, scale, 0)
                pltpu.async_copy(gbuf, acc.at[relb.at[0]], sem,
                                 add=True).wait()
                return 0
            lax.fori_loop(0, ng, group, 0)
            base = ng * 128
            for i in range(8):
                t1 = crel[pl.ds(base + i * 16, 16)]
                t2 = cgid[pl.ds(base + i * 16, 16)]
                t3 = cval[pl.ds(base + i * 16, 16)]
                crel[pl.ds(i * 16, 16)] = t1
                cgid[pl.ds(i * 16, 16)] = t2
                cval[pl.ds(i * 16, 16)] = t3
            return off - base

        def item_body(win, _):
            wlo = win * win_r

            def zero_blk(z):
                pltpu.sync_copy(zbuf, acc.at[pl.ds(sid * tile_rows + z * zc, zc)])
            pl.loop(0, tile_rows // zc)(zero_blk)
            plsc.subcore_barrier()

            def blk_body(b, off):
                o = sid * tile_nnz + b * BLK
                d1 = pltpu.async_copy(rows_hbm.at[pl.ds(o, BLK)], rows_v, sem)
                d2 = pltpu.async_copy(cols_hbm.at[pl.ds(o, BLK)], cols_v, sem)
                d3 = pltpu.async_copy(vals_hbm.at[pl.ds(o, BLK)], vals_v, sem)
                d1.wait(); d2.wait(); d3.wait()

                def chunk(j, off):
                    r16 = rows_v[pl.ds(j * 16, 16)]
                    c16 = cols_v[pl.ds(j * 16, 16)]
                    v16 = vals_v[pl.ds(j * 16, 16)]
                    inwin = (r16 >= wlo) & (r16 < wlo + win_r)
                    cum = _prefix16(inwin, iota16)
                    pos = jnp.where(inwin, off + cum - 1, zeros16 + (CBUF - 1))
                    plsc.store_scatter(crel, [pos], r16 - wlo)
                    plsc.store_scatter(cgid, [pos], c16 * 2 + cid)
                    plsc.store_scatter(cval, [pos], v16)
                    return off + cum[15]
                off = lax.fori_loop(0, BLK // 16, chunk, off)
                return drain(off)
            off = lax.fori_loop(0, n_blk, blk_body, jnp.int32(0))

            for i in range(8):
                crel[pl.ds(off + i * 16, 16)] = zeros16 + win_r
                cgid[pl.ds(off + i * 16, 16)] = zeros16
                cval[pl.ds(off + i * 16, 16)] = jnp.zeros((16,), jnp.float32)
            drain(((off + 127) // 128) * 128)
            plsc.subcore_barrier()

            pltpu.sync_copy(
                acc.at[pl.ds(sid * tile_rows, tile_rows)],
                out_hbm.at[pl.ds(wlo + sid * tile_rows, tile_rows), cid, :])
            plsc.subcore_barrier()
            return 0
        lax.fori_loop(0, n_win, item_body, 0)

    return spmm


def _make_spmm(nnz, r_out, n_win, zc):
    win_r = r_out // n_win
    items_per_sc = (n_win * NITEMS) // NCORES
    tile_rows = win_r // NSUB
    tile_nnz = nnz // NSUB
    n_blk = tile_nnz // BLK
    n_row = BLK // 128

    mesh = plsc.VectorSubcoreMesh(core_axis_name="c", subcore_axis_name="s",
                                  num_cores=NCORES, num_subcores=NSUB)

    @functools.partial(
        pl.kernel,
        out_type=jax.ShapeDtypeStruct((r_out, NITEMS, FW), jnp.float32),
        mesh=mesh,
        scratch_types=[
            pltpu.VMEM_SHARED((win_r + 16, FW), jnp.float32),
            pltpu.VMEM((BLK,), jnp.int32),
            pltpu.VMEM((BLK,), jnp.int32),
            pltpu.VMEM((BLK,), jnp.float32),
            pltpu.VMEM((n_row, 128), jnp.int32),
            pltpu.VMEM((n_row, 128), jnp.int32),
            pltpu.VMEM((BLK, FW), jnp.float32),
            pltpu.VMEM((zc, FW), jnp.float32),
            pltpu.SemaphoreType.DMA,
        ],
        compiler_params=pltpu.CompilerParams(use_tc_tiling_on_sc=False, needs_layout_passes=False),
    )
    def spmm(x_hbm, rows_hbm, cols_hbm, vals_hbm, out_hbm,
             acc, rows_v, cols_v, vals_v, idxb, relb, gbuf, zbuf, sem):
        cid = lax.axis_index("c")
        sid = lax.axis_index("s")

        def zinit(zr):
            zbuf[zr, :] = jnp.zeros((FW,), jnp.float32)
        pl.loop(0, zc)(zinit)

        def item_body(it):
            g_it = cid * items_per_sc + it
            win = g_it // NITEMS
            fch = g_it % NITEMS
            wlo = win * win_r

            def zero_blk(z):
                pltpu.sync_copy(zbuf, acc.at[pl.ds(sid * tile_rows + z * zc, zc)])
            pl.loop(0, tile_rows // zc)(zero_blk)
            plsc.subcore_barrier()

            def blk_body(b):
                off = sid * tile_nnz + b * BLK
                d1 = pltpu.async_copy(rows_hbm.at[pl.ds(off, BLK)], rows_v, sem)
                d2 = pltpu.async_copy(cols_hbm.at[pl.ds(off, BLK)], cols_v, sem)
                d3 = pltpu.async_copy(vals_hbm.at[pl.ds(off, BLK)], vals_v, sem)
                d1.wait(); d2.wait(); d3.wait()

                def idx_body(r):
                    def idx16(q):
                        j = r * 8 + q
                        c16 = cols_v[pl.ds(j * 16, 16)]
                        r16 = rows_v[pl.ds(j * 16, 16)]
                        inwin = (r16 >= wlo) & (r16 < wlo + win_r)
                        rel = jnp.where(inwin, r16 - wlo, win_r)
                        v16 = vals_v[pl.ds(j * 16, 16)]
                        vals_v[pl.ds(j * 16, 16)] = jnp.where(
                            inwin, v16, jnp.zeros((16,), jnp.float32))
                        idxb[r, pl.ds(q * 16, 16)] = c16 * NITEMS + fch
                        relb[r, pl.ds(q * 16, 16)] = rel
                    pl.loop(0, 8)(idx16)
                pl.loop(0, n_row)(idx_body)

                gd = [pltpu.async_copy(x_hbm.at[idxb.at[r]],
                                       gbuf.at[pl.ds(r * 128, 128)], sem)
                      for r in range(n_row)]
                for d in gd:
                    d.wait()

                zeros16 = lax.broadcasted_iota(jnp.int32, (16,), 0) * 0
                def scale16(j):
                    v16 = vals_v[pl.ds(j * 16, 16)]
                    for i in range(16):
                        e = j * 16 + i
                        gbuf[e, :] = gbuf[e, :] * _splat(v16, zeros16, i)
                pl.loop(0, BLK // 16)(scale16)

                sd = [pltpu.async_copy(gbuf.at[pl.ds(r * 128, 128)],
                                       acc.at[relb.at[r]], sem, add=True)
                      for r in range(n_row)]
                for d in sd:
                    d.wait()
            pl.loop(0, n_blk)(blk_body)
            plsc.subcore_barrier()

            pltpu.sync_copy(
                acc.at[pl.ds(sid * tile_rows, tile_rows)],
                out_hbm.at[pl.ds(wlo + sid * tile_rows, tile_rows), fch, :])
            plsc.subcore_barrier()
        pl.loop(0, items_per_sc)(item_body)

    return spmm


_spmm_g = _make_spmm2(737280, 3 * NF, 12, 128)
_spmm_nv = _make_spmm(294912, NVP, 1, 432)
_spmm_f2v = _make_spmm(245760, NVP, 1, 432)

_NF_BLK = 512
_NV_BLK = 512


def _k2_body(y3, ewt, nst, ew_o, ns_o):
    y = y3[...]
    ew = ewt[...]
    ns = nst[...]
    ew_o[...] = (y[0] * ew[0][:, None] + y[1] * ew[1][:, None]
                 + y[2] * ew[2][:, None])
    ns_o[...] = (y[0] * ns[0][:, None] + y[1] * ns[1][:, None]
                 + y[2] * ns[2][:, None])


def _k4_body(xv, fl, fe, fn, m2, b2, out):
    acc = jnp.dot(xv[...], m2[0], preferred_element_type=jnp.float32)
    acc += jnp.dot(fl[...], m2[1], preferred_element_type=jnp.float32)
    acc += jnp.dot(fe[...], m2[2], preferred_element_type=jnp.float32)
    acc += jnp.dot(fn[...], m2[3], preferred_element_type=jnp.float32)
    out[...] = acc + b2[...]


def kernel(x, G_rows, G_cols, G_vals, L_rows, L_cols, L_vals,
           F2V_rows, F2V_cols, F2V_vals, NS, EW, coeffs, bias):
    x_vm = jnp.pad(x.reshape(B * C, NV).T, ((0, NVP - NV), (0, 0)))
    x_flat = x_vm.reshape(NVP * NITEMS, FW)
    lp = 294912 - L_rows.shape[0]
    l_rows = jnp.pad(L_rows, (0, lp))
    l_cols = jnp.pad(L_cols, (0, lp))
    l_vals = jnp.pad(L_vals, (0, lp))

    x_flat2 = x_vm.reshape(NVP * 2, 64)
    y3 = _spmm_g(x_flat2, G_rows, G_cols, G_vals).reshape(3, NF, 128)

    f_lap = _spmm_nv(x_flat, l_rows, l_cols, l_vals).reshape(NVP, 128)

    ewt = EW.T
    nst = NS.T
    gf_ew, gf_ns = pl.pallas_call(
        _k2_body,
        grid=(NF // _NF_BLK,),
        in_specs=[
            pl.BlockSpec((3, _NF_BLK, 128), lambda i: (0, i, 0)),
            pl.BlockSpec((3, _NF_BLK), lambda i: (0, i)),
            pl.BlockSpec((3, _NF_BLK), lambda i: (0, i)),
        ],
        out_specs=[
            pl.BlockSpec((_NF_BLK, 128), lambda i: (i, 0)),
            pl.BlockSpec((_NF_BLK, 128), lambda i: (i, 0)),
        ],
        out_shape=[jax.ShapeDtypeStruct((NF, 128), jnp.float32)] * 2,
    )(y3, ewt, nst)

    f_ew = _spmm_f2v(gf_ew.reshape(NF * NITEMS, FW),
                     F2V_rows, F2V_cols, F2V_vals).reshape(NVP, 128)
    f_ns = _spmm_f2v(gf_ns.reshape(NF * NITEMS, FW),
                     F2V_rows, F2V_cols, F2V_vals).reshape(NVP, 128)

    a = jnp.transpose(coeffs, (2, 1, 0))
    m2 = jnp.zeros((4, 128, 128), jnp.float32)
    m2 = m2.at[:, :C, :C].set(a).at[:, C:, C:].set(a)
    b2 = jnp.concatenate([bias, bias])[None]
    out = pl.pallas_call(
        _k4_body,
        grid=(NVP // _NV_BLK,),
        in_specs=[
            pl.BlockSpec((_NV_BLK, 128), lambda i: (i, 0)),
            pl.BlockSpec((_NV_BLK, 128), lambda i: (i, 0)),
            pl.BlockSpec((_NV_BLK, 128), lambda i: (i, 0)),
            pl.BlockSpec((_NV_BLK, 128), lambda i: (i, 0)),
            pl.BlockSpec((4, 128, 128), lambda i: (0, 0, 0)),
            pl.BlockSpec((1, 128), lambda i: (0, 0)),
        ],
        out_specs=pl.BlockSpec((_NV_BLK, 128), lambda i: (i, 0)),
        out_shape=jax.ShapeDtypeStruct((NVP, 128), jnp.float32),
    )(x_vm, f_lap, f_ew, f_ns, m2, b2)

    return out[:NV].T.reshape(B, C, NV)
